# baseline (speedup 1.0000x reference)
import jax
import jax.numpy as jnp
from jax import lax
from jax.experimental import pallas as pl
from jax.experimental.pallas import tpu as pltpu
from jax.experimental.pallas import tpu_sc as plsc

VOCAB = 1000000
NUM_BUCKETS = 10
EMB = 64
BATCH = 16384
OUT_COLS = 2 * EMB + 1

NC, NS, L = 2, 16, 16
NW = NC * NS
BPW = BATCH // NW
GCHUNK = 128


def _body(uid_hbm, age_hbm, table_hbm, agetabt_hbm, buckets_hbm, mscale_hbm,
          out_hbm,
          uid_v, agev_v, agetabt_v, urows_v, obuf, buckets_v, mscale_v, sem):
    wid = lax.axis_index("s") * NC + lax.axis_index("c")
    base = wid * BPW

    pltpu.sync_copy(uid_hbm.at[pl.ds(base, BPW)], uid_v)
    pltpu.sync_copy(age_hbm.at[pl.ds(base, BPW)], agev_v)
    pltpu.sync_copy(buckets_hbm, buckets_v)
    pltpu.sync_copy(mscale_hbm, mscale_v)
    pltpu.sync_copy(agetabt_hbm, agetabt_v)

    mean = mscale_v[0]
    scale = mscale_v[1]

    def shift(g, carry):
        sl = pl.ds(g * L, L)
        uid_v[sl] = uid_v[sl] + 1
        return carry

    lax.fori_loop(0, BPW // L, shift, 0)

    copies = []
    for k in range(BPW // GCHUNK):
        sl = pl.ds(k * GCHUNK, GCHUNK)
        copies.append(pltpu.async_copy(table_hbm.at[uid_v.at[sl]],
                                       urows_v.at[sl], sem))

    lanes = lax.iota(jnp.int32, L)

    def agefill(g, carry):
        sl = pl.ds(g * L, L)
        a = agev_v[sl]
        one = jnp.ones((L,), jnp.int32)
        zero = jnp.zeros((L,), jnp.int32)
        cnt = jnp.where(a >= buckets_v[0], one, zero)
        for j in range(1, NUM_BUCKETS):
            cnt = cnt + jnp.where(a >= buckets_v[j], one, zero)
        obuf[2 * EMB, sl] = (a - mean) * scale
        for c in range(EMB):
            col = jnp.full((L,), c, jnp.int32)
            obuf[EMB + c, sl] = plsc.load_gather(agetabt_v, [col, cnt])
        return carry

    lax.fori_loop(0, BPW // L, agefill, 0)

    for c in copies:
        c.wait()

    def mv(g, carry):
        rows = lanes + g * L
        for c in range(EMB):
            col = jnp.full((L,), c, jnp.int32)
            obuf[c, pl.ds(g * L, L)] = plsc.load_gather(urows_v, [rows, col])
        return carry

    lax.fori_loop(0, BPW // L, mv, 0)

    pltpu.sync_copy(obuf, out_hbm.at[:, pl.ds(base, BPW)])


@jax.jit
def _sc_lookup(user_id, age, user_table, age_table_t, buckets_b, mscale):
    mesh = plsc.VectorSubcoreMesh(core_axis_name="c", subcore_axis_name="s")
    f = pl.kernel(
        _body,
        out_type=jax.ShapeDtypeStruct((OUT_COLS, BATCH), jnp.float32),
        mesh=mesh,
        scratch_types=[
            pltpu.VMEM((BPW,), jnp.int32),
            pltpu.VMEM((BPW,), jnp.float32),
            pltpu.VMEM((EMB, NUM_BUCKETS + 1), jnp.float32),
            pltpu.VMEM((BPW, EMB), jnp.float32),
            pltpu.VMEM((OUT_COLS, BPW), jnp.float32),
            pltpu.VMEM((NUM_BUCKETS, L), jnp.float32),
            pltpu.VMEM((2, L), jnp.float32),
            pltpu.SemaphoreType.DMA,
        ],
        compiler_params=pltpu.CompilerParams(use_tc_tiling_on_sc=False,
                                             needs_layout_passes=False),
    )
    return f(user_id, age, user_table, age_table_t, buckets_b, mscale)


def kernel(user_id, age, user_table, age_table, age_buckets, age_mean, age_var):
    buckets_b = jnp.broadcast_to(age_buckets[:, None], (NUM_BUCKETS, L))
    scale = lax.rsqrt(age_var.astype(jnp.float32))
    mscale = jnp.stack([jnp.full((L,), age_mean, jnp.float32),
                        jnp.full((L,), scale, jnp.float32)])
    out_t = _sc_lookup(user_id, age, user_table, jnp.transpose(age_table),
                       buckets_b, mscale)
    return out_t.T


# speedup vs baseline: 1.6645x; 1.6645x over previous
import jax
import jax.numpy as jnp
from jax import lax
from jax.experimental import pallas as pl
from jax.experimental.pallas import tpu as pltpu
from jax.experimental.pallas import tpu_sc as plsc

VOCAB = 1000000
NUM_BUCKETS = 10
EMB = 64
BATCH = 16384
OUT_COLS = 2 * EMB + 1

NC, NS, L = 2, 16, 16
NW = NC * NS
BPW = BATCH // NW
HALF = BPW // 2
GCHUNK = 128


def _body(uid_hbm, age_hbm, table_hbm, agetabt_hbm, buckets_hbm, mscale_hbm,
          out_hbm,
          uid_v, agev_v, agetabt_v, urows_v, obuf, buckets_v, mscale_v, sem):
    wid = lax.axis_index("s") * NC + lax.axis_index("c")
    base = wid * BPW

    pltpu.sync_copy(uid_hbm.at[pl.ds(base, BPW)], uid_v)
    pltpu.sync_copy(age_hbm.at[pl.ds(base, BPW)], agev_v)
    pltpu.sync_copy(buckets_hbm, buckets_v)
    pltpu.sync_copy(mscale_hbm, mscale_v)
    pltpu.sync_copy(agetabt_hbm, agetabt_v)

    mean = mscale_v[0]
    scale = mscale_v[1]

    def shift(g, carry):
        sl = pl.ds(g * L, L)
        uid_v[sl] = uid_v[sl] + 1
        return carry

    lax.fori_loop(0, BPW // L, shift, 0)

    lanes = lax.iota(jnp.int32, L)

    def fetch(h):
        def body(g, carry):
            uvec = uid_v[pl.ds(h * HALF + g * L, L)]
            for l in range(L):
                u = uvec[l]
                pltpu.async_copy(table_hbm.at[pl.ds(u, 1)],
                                 urows_v.at[pl.ds(g * L + l, 1)], sem)
            return carry

        lax.fori_loop(0, HALF // L, body, 0)

    def drain():
        pltpu.make_async_copy(table_hbm.at[pl.ds(0, HALF)], urows_v,
                              sem).wait()

    def mv(h):
        def body(g, carry):
            rows = lanes + g * L
            for c in range(EMB):
                col = jnp.full((L,), c, jnp.int32)
                obuf[c, pl.ds(h * HALF + g * L, L)] = plsc.load_gather(
                    urows_v, [rows, col])
            return carry

        lax.fori_loop(0, HALF // L, body, 0)

    fetch(0)

    def agefill(g, carry):
        sl = pl.ds(g * L, L)
        a = agev_v[sl]
        one = jnp.ones((L,), jnp.int32)
        zero = jnp.zeros((L,), jnp.int32)
        cnt = jnp.where(a >= buckets_v[0], one, zero)
        for j in range(1, NUM_BUCKETS):
            cnt = cnt + jnp.where(a >= buckets_v[j], one, zero)
        obuf[2 * EMB, sl] = (a - mean) * scale
        for c in range(EMB):
            col = jnp.full((L,), c, jnp.int32)
            obuf[EMB + c, sl] = plsc.load_gather(agetabt_v, [col, cnt])
        return carry

    lax.fori_loop(0, BPW // L, agefill, 0)

    drain()
    mv(0)
    fetch(1)
    drain()
    mv(1)

    pltpu.sync_copy(obuf, out_hbm.at[:, pl.ds(base, BPW)])


@jax.jit
def _sc_lookup(user_id, age, user_table, age_table_t, buckets_b, mscale):
    mesh = plsc.VectorSubcoreMesh(core_axis_name="c", subcore_axis_name="s")
    f = pl.kernel(
        _body,
        out_type=jax.ShapeDtypeStruct((OUT_COLS, BATCH), jnp.float32),
        mesh=mesh,
        scratch_types=[
            pltpu.VMEM((BPW,), jnp.int32),
            pltpu.VMEM((BPW,), jnp.float32),
            pltpu.VMEM((EMB, NUM_BUCKETS + 1), jnp.float32),
            pltpu.VMEM((HALF, EMB), jnp.float32),
            pltpu.VMEM((OUT_COLS, BPW), jnp.float32),
            pltpu.VMEM((NUM_BUCKETS, L), jnp.float32),
            pltpu.VMEM((2, L), jnp.float32),
            pltpu.SemaphoreType.DMA,
        ],
        compiler_params=pltpu.CompilerParams(needs_layout_passes=False),
    )
    return f(user_id, age, user_table, age_table_t, buckets_b, mscale)


def kernel(user_id, age, user_table, age_table, age_buckets, age_mean, age_var):
    buckets_b = jnp.broadcast_to(age_buckets[:, None], (NUM_BUCKETS, L))
    scale = lax.rsqrt(age_var.astype(jnp.float32))
    mscale = jnp.stack([jnp.full((L,), age_mean, jnp.float32),
                        jnp.full((L,), scale, jnp.float32)])
    out_t = _sc_lookup(user_id, age, user_table, jnp.transpose(age_table),
                       buckets_b, mscale)
    return out_t.T
